# Initial kernel scaffold; baseline (speedup 1.0000x reference)
import jax
import jax.numpy as jnp
from jax.experimental import pallas as pl


def kernel(x, edge_index, fc1_w, fc1_b, Ws, fc2_w, fc2_b):
    raise NotImplementedError("write your pallas kernel here")


# speedup vs baseline: 2.3698x; 2.3698x over previous
import functools
import math

import jax
import jax.numpy as jnp
from jax import lax
from jax.experimental import pallas as pl
from jax.experimental.pallas import tpu as pltpu
from jax.experimental.pallas import tpu_sc as plsc

N = 10000
E = 320000
D = 128
DC = 64
NLAYERS = 4
ALPHA_C = 0.5
LAMB_C = 1.0

NC = 2
NS = 16
NW = NC * NS
CH = 128
CPT = 80
EP = NW * CPT * CH
PAD = EP - E
AGG_ROWS = 10112
ZPT = AGG_ROWS // NS

DEG_ROWS_TOTAL = EP // CH
DEG_RPT = DEG_ROWS_TOTAL // NS

BM = 1000
GRID = N // BM


def _sc_mesh():
    return plsc.VectorSubcoreMesh(core_axis_name="c", subcore_axis_name="s")


def _sc_degrees(ei_flat):
    @functools.partial(
        pl.kernel,
        out_type=jax.ShapeDtypeStruct((NC * AGG_ROWS, 16), jnp.float32),
        mesh=_sc_mesh(),
        scratch_types=[
            pltpu.VMEM((DEG_RPT, CH), jnp.int32),
            pltpu.VMEM((CH, 16), jnp.float32),
            pltpu.VMEM((CH, 16), jnp.float32),
            pltpu.VMEM_SHARED((AGG_ROWS, 16), jnp.float32),
        ],
    )
    def k(ei_hbm, out_hbm, idx_v, ones_v, zero_v, hist_sh):
        c = lax.axis_index("c")
        s = lax.axis_index("s")

        @pl.loop(0, CH)
        def _(i):
            ones_v[i, pl.ds(0, 16)] = jnp.ones((16,), jnp.float32)
            zero_v[i, pl.ds(0, 16)] = jnp.zeros((16,), jnp.float32)

        zbase = s * ZPT
        for kk in range(ZPT // CH):
            pltpu.sync_copy(zero_v, hist_sh.at[pl.ds(zbase + kk * CH, CH)])
        rem = ZPT % CH
        pltpu.sync_copy(zero_v.at[pl.ds(0, rem)],
                        hist_sh.at[pl.ds(zbase + (ZPT // CH) * CH, rem)])

        pltpu.sync_copy(
            ei_hbm.at[pl.ds(c * DEG_ROWS_TOTAL + s * DEG_RPT, DEG_RPT)], idx_v)
        plsc.subcore_barrier()

        @pl.loop(0, DEG_RPT)
        def _(j):
            pltpu.sync_copy(ones_v, hist_sh.at[idx_v.at[j]], add=True)

        plsc.subcore_barrier()
        pltpu.sync_copy(hist_sh.at[pl.ds(zbase, ZPT)],
                        out_hbm.at[pl.ds(c * AGG_ROWS + zbase, ZPT)])

    return k(ei_flat)


def _sc_propagate(hs, src2d, dst2d):
    @functools.partial(
        pl.kernel,
        out_type=jax.ShapeDtypeStruct((NC * AGG_ROWS, D), jnp.float32),
        mesh=_sc_mesh(),
        scratch_types=[
            pltpu.VMEM((CPT, CH), jnp.int32),
            pltpu.VMEM((CPT, CH), jnp.int32),
            pltpu.VMEM((CH, D), jnp.float32),
            pltpu.VMEM_SHARED((AGG_ROWS, D), jnp.float32),
            pltpu.SemaphoreType.DMA,
        ],
    )
    def k(hs_hbm, src_hbm, dst_hbm, out_hbm, src_v, dst_v, buf, agg_sh, sem):
        c = lax.axis_index("c")
        s = lax.axis_index("s")
        wid = c * NS + s

        @pl.loop(0, CH)
        def _(i):
            for kk in range(D // 16):
                buf[i, pl.ds(kk * 16, 16)] = jnp.zeros((16,), jnp.float32)

        zbase = s * ZPT
        for kk in range(ZPT // CH):
            pltpu.sync_copy(buf, agg_sh.at[pl.ds(zbase + kk * CH, CH)])
        rem = ZPT % CH
        pltpu.sync_copy(buf.at[pl.ds(0, rem)],
                        agg_sh.at[pl.ds(zbase + (ZPT // CH) * CH, rem)])

        row0 = wid * CPT
        pltpu.sync_copy(src_hbm.at[pl.ds(row0, CPT)], src_v)
        pltpu.sync_copy(dst_hbm.at[pl.ds(row0, CPT)], dst_v)
        plsc.subcore_barrier()

        @pl.loop(0, CPT)
        def _(j):
            pltpu.async_copy(hs_hbm.at[src_v.at[j]], buf, sem).wait()
            pltpu.sync_copy(buf, agg_sh.at[dst_v.at[j]], add=True)

        plsc.subcore_barrier()
        pltpu.sync_copy(agg_sh.at[pl.ds(zbase, ZPT)],
                        out_hbm.at[pl.ds(c * AGG_ROWS + zbase, ZPT)])

    return k(hs, src2d, dst2d)


def _fc1_body(x_ref, w_ref, b_ref, dout_ref, h0_ref, hs0_ref):
    acc = jnp.dot(x_ref[...], w_ref[...], preferred_element_type=jnp.float32)
    h0 = jnp.maximum(acc + b_ref[0][None, :], 0.0)
    ns = lax.rsqrt(jnp.maximum(dout_ref[...], 1.0))
    h0_ref[...] = h0
    hs0_ref[...] = h0 * ns


def _tc_fc1(x, fc1_w, fc1_b, deg_out):
    return pl.pallas_call(
        _fc1_body,
        grid=(GRID,),
        in_specs=[
            pl.BlockSpec((BM, D), lambda i: (i, 0)),
            pl.BlockSpec((D, D), lambda i: (0, 0)),
            pl.BlockSpec((1, D), lambda i: (0, 0)),
            pl.BlockSpec((BM, 1), lambda i: (i, 0)),
        ],
        out_specs=[
            pl.BlockSpec((BM, D), lambda i: (i, 0)),
            pl.BlockSpec((BM, D), lambda i: (i, 0)),
        ],
        out_shape=[
            jax.ShapeDtypeStruct((N, D), jnp.float32),
            jax.ShapeDtypeStruct((N, D), jnp.float32),
        ],
    )(x, fc1_w, fc1_b, deg_out)


def _layer_body(beta, p0_ref, p1_ref, h0_ref, h_ref, din_ref, dout_ref, w_ref,
                hn_ref, hsn_ref):
    nd = lax.rsqrt(jnp.maximum(din_ref[...], 1.0))
    feat = (p0_ref[...] + p1_ref[...]) * nd
    feat = (1.0 - ALPHA_C) * feat + ALPHA_C * h0_ref[...]
    mm = jnp.dot(feat, w_ref[...], preferred_element_type=jnp.float32)
    rst = (1.0 - beta) * feat + beta * mm
    hn = jnp.maximum(rst + h_ref[...], 0.0)
    ns = lax.rsqrt(jnp.maximum(dout_ref[...], 1.0))
    hn_ref[...] = hn
    hsn_ref[...] = hn * ns


def _tc_layer(beta, p0, p1, h0, h, deg_in, deg_out, w):
    return pl.pallas_call(
        functools.partial(_layer_body, beta),
        grid=(GRID,),
        in_specs=[
            pl.BlockSpec((BM, D), lambda i: (i, 0)),
            pl.BlockSpec((BM, D), lambda i: (i, 0)),
            pl.BlockSpec((BM, D), lambda i: (i, 0)),
            pl.BlockSpec((BM, D), lambda i: (i, 0)),
            pl.BlockSpec((BM, 1), lambda i: (i, 0)),
            pl.BlockSpec((BM, 1), lambda i: (i, 0)),
            pl.BlockSpec((D, D), lambda i: (0, 0)),
        ],
        out_specs=[
            pl.BlockSpec((BM, D), lambda i: (i, 0)),
            pl.BlockSpec((BM, D), lambda i: (i, 0)),
        ],
        out_shape=[
            jax.ShapeDtypeStruct((N, D), jnp.float32),
            jax.ShapeDtypeStruct((N, D), jnp.float32),
        ],
    )(p0, p1, h0, h, deg_in, deg_out, w)


def _fc2_body(h_ref, w_ref, b_ref, o_ref):
    acc = jnp.dot(h_ref[...], w_ref[...], preferred_element_type=jnp.float32)
    o_ref[...] = acc + b_ref[0][None, :]


def _tc_fc2(h, fc2_w, fc2_b):
    return pl.pallas_call(
        _fc2_body,
        grid=(GRID,),
        in_specs=[
            pl.BlockSpec((BM, D), lambda i: (i, 0)),
            pl.BlockSpec((D, DC), lambda i: (0, 0)),
            pl.BlockSpec((1, DC), lambda i: (0, 0)),
        ],
        out_specs=pl.BlockSpec((BM, DC), lambda i: (i, 0)),
        out_shape=jax.ShapeDtypeStruct((N, DC), jnp.float32),
    )(h, fc2_w, fc2_b)


def _sc_copy(xin):
    nrows = xin.shape[0]
    rpw = nrows // NW

    @functools.partial(
        pl.kernel,
        out_type=jax.ShapeDtypeStruct(xin.shape, jnp.float32),
        mesh=_sc_mesh(),
        scratch_types=[
            pltpu.VMEM((rpw, 128), jnp.float32),
            pltpu.VMEM_SHARED((nrows, 128), jnp.float32),
        ],
    )
    def k(in_hbm, out_hbm, buf, sh):
        c = lax.axis_index("c")
        s = lax.axis_index("s")
        wid = c * NS + s

        pltpu.sync_copy(in_hbm.at[pl.ds(wid * rpw, rpw)], buf)
        pltpu.sync_copy(buf, sh.at[pl.ds(wid * rpw, rpw)])
        pltpu.sync_copy(sh.at[pl.ds(wid * rpw, rpw)],
                        out_hbm.at[pl.ds(wid * rpw, rpw)])

    return k(xin)


def kernel(x, edge_index, fc1_w, fc1_b, Ws, fc2_w, fc2_b):
    src = edge_index[0]
    dst = edge_index[1]

    src_p = jnp.concatenate([src, jnp.zeros((PAD,), jnp.int32)]).reshape(
        NW * CPT, CH)
    dst_p = jnp.concatenate([dst, jnp.full((PAD,), N, jnp.int32)]).reshape(
        NW * CPT, CH)
    ei_deg = jnp.concatenate(
        [edge_index, jnp.full((2, PAD), N, jnp.int32)], axis=1).reshape(
        2 * DEG_ROWS_TOTAL, CH)

    dj_out = jnp.bincount(src, length=N).astype(jnp.float32)
    dj_in = jnp.bincount(dst, length=N).astype(jnp.float32)
    dcat = jnp.broadcast_to(
        jnp.concatenate([dj_out, dj_in])[:, None], (2 * N, 16))
    dcat = jnp.concatenate(
        [dcat, jnp.zeros((480, 16), jnp.float32)]).reshape(2560, 128)
    degs = _sc_copy(dcat).reshape(20480, 16)
    deg_out = degs[:N, 0:1]
    deg_in = degs[N:2 * N, 0:1]

    h0, hs = _tc_fc1(x, fc1_w, fc1_b.reshape(1, D), deg_out)
    h = h0
    for l in range(NLAYERS):
        part = _sc_propagate(hs, src_p, dst_p)
        beta = math.log(LAMB_C / (l + 1) + 1.0)
        h, hs = _tc_layer(beta, part[:N], part[AGG_ROWS:AGG_ROWS + N], h0, h,
                          deg_in, deg_out, Ws[l])
    return _tc_fc2(h, fc2_w, fc2_b.reshape(1, DC))


# speedup vs baseline: 2.5406x; 1.0721x over previous
import functools
import math

import jax
import jax.numpy as jnp
from jax import lax
from jax.experimental import pallas as pl
from jax.experimental.pallas import tpu as pltpu
from jax.experimental.pallas import tpu_sc as plsc

N = 10000
E = 320000
D = 128
DC = 64
NLAYERS = 4
ALPHA_C = 0.5
LAMB_C = 1.0

NC = 2
NS = 16
NW = NC * NS
CH = 128
CPT = 80
WIN = 40
EP = NW * CPT * CH
PAD = EP - E
AGG_ROWS = 10112
ZPT = AGG_ROWS // NS

DEG_ROWS_TOTAL = EP // CH
DEG_RPT = DEG_ROWS_TOTAL // NS

BM = 1000
GRID = N // BM


def _sc_mesh():
    return plsc.VectorSubcoreMesh(core_axis_name="c", subcore_axis_name="s")


def _sc_degrees(ei_flat):
    @functools.partial(
        pl.kernel,
        out_type=jax.ShapeDtypeStruct((NC * AGG_ROWS, 16), jnp.float32),
        mesh=_sc_mesh(),
        scratch_types=[
            pltpu.VMEM((DEG_RPT, CH), jnp.int32),
            pltpu.VMEM((CH, 16), jnp.float32),
            pltpu.VMEM((CH, 16), jnp.float32),
            pltpu.VMEM_SHARED((AGG_ROWS, 16), jnp.float32),
        ],
    )
    def k(ei_hbm, out_hbm, idx_v, ones_v, zero_v, hist_sh):
        c = lax.axis_index("c")
        s = lax.axis_index("s")

        @pl.loop(0, CH)
        def _(i):
            ones_v[i, pl.ds(0, 16)] = jnp.ones((16,), jnp.float32)
            zero_v[i, pl.ds(0, 16)] = jnp.zeros((16,), jnp.float32)

        zbase = s * ZPT
        for kk in range(ZPT // CH):
            pltpu.sync_copy(zero_v, hist_sh.at[pl.ds(zbase + kk * CH, CH)])
        rem = ZPT % CH
        pltpu.sync_copy(zero_v.at[pl.ds(0, rem)],
                        hist_sh.at[pl.ds(zbase + (ZPT // CH) * CH, rem)])

        pltpu.sync_copy(
            ei_hbm.at[pl.ds(c * DEG_ROWS_TOTAL + s * DEG_RPT, DEG_RPT)], idx_v)
        plsc.subcore_barrier()

        @pl.loop(0, DEG_RPT)
        def _(j):
            pltpu.sync_copy(ones_v, hist_sh.at[idx_v.at[j]], add=True)

        plsc.subcore_barrier()
        pltpu.sync_copy(hist_sh.at[pl.ds(zbase, ZPT)],
                        out_hbm.at[pl.ds(c * AGG_ROWS + zbase, ZPT)])

    return k(ei_flat)


def _sc_propagate(hs, src2d, dst2d):
    @functools.partial(
        pl.kernel,
        out_type=jax.ShapeDtypeStruct((NC * AGG_ROWS, D), jnp.float32),
        mesh=_sc_mesh(),
        scratch_types=[
            pltpu.VMEM((WIN, CH), jnp.int32),
            pltpu.VMEM((WIN, CH), jnp.int32),
            pltpu.VMEM((CH, D), jnp.float32),
            pltpu.VMEM((CH, D), jnp.float32),
            pltpu.VMEM_SHARED((AGG_ROWS, D), jnp.float32),
            pltpu.SemaphoreType.DMA,
            pltpu.SemaphoreType.DMA,
        ],
    )
    def k(hs_hbm, src_hbm, dst_hbm, out_hbm, src_v, dst_v, bufa, bufb, agg_sh,
          sema, semb):
        c = lax.axis_index("c")
        s = lax.axis_index("s")
        wid = c * NS + s

        @pl.loop(0, CH)
        def _(i):
            for kk in range(D // 16):
                bufa[i, pl.ds(kk * 16, 16)] = jnp.zeros((16,), jnp.float32)

        zbase = s * ZPT
        for kk in range(ZPT // CH):
            pltpu.sync_copy(bufa, agg_sh.at[pl.ds(zbase + kk * CH, CH)])
        rem = ZPT % CH
        pltpu.sync_copy(bufa.at[pl.ds(0, rem)],
                        agg_sh.at[pl.ds(zbase + (ZPT // CH) * CH, rem)])

        row0 = wid * CPT
        pltpu.sync_copy(src_hbm.at[pl.ds(row0, WIN)], src_v)
        pltpu.sync_copy(dst_hbm.at[pl.ds(row0, WIN)], dst_v)
        plsc.subcore_barrier()

        def wrow(k):
            return k - jnp.where(k >= WIN, WIN, 0)

        def start_g(k, buf, sem):
            pltpu.async_copy(hs_hbm.at[src_v.at[wrow(k)]], buf, sem)

        def wait_g(buf, sem):
            pltpu.make_async_copy(hs_hbm.at[src_v.at[0]], buf, sem).wait()

        def scat(k, buf):
            pltpu.sync_copy(buf, agg_sh.at[dst_v.at[wrow(k)]], add=True)

        start_g(0, bufa, sema)

        @pl.loop(0, CPT - 2, step=2)
        def _(j):
            wait_g(bufa, sema)
            start_g(j + 1, bufb, semb)
            scat(j, bufa)
            wait_g(bufb, semb)

            @pl.when(j == WIN - 2)
            def _():
                pltpu.sync_copy(src_hbm.at[pl.ds(row0 + WIN, WIN)], src_v)

            start_g(j + 2, bufa, sema)
            scat(j + 1, bufb)

            @pl.when(j == WIN - 2)
            def _():
                pltpu.sync_copy(dst_hbm.at[pl.ds(row0 + WIN, WIN)], dst_v)

        wait_g(bufa, sema)
        start_g(CPT - 1, bufb, semb)
        scat(CPT - 2, bufa)
        wait_g(bufb, semb)
        scat(CPT - 1, bufb)

        plsc.subcore_barrier()
        pltpu.sync_copy(agg_sh.at[pl.ds(zbase, ZPT)],
                        out_hbm.at[pl.ds(c * AGG_ROWS + zbase, ZPT)])

    return k(hs, src2d, dst2d)


def _fc1_body(x_ref, w_ref, b_ref, dout_ref, h0_ref, hs0_ref):
    acc = jnp.dot(x_ref[...], w_ref[...], preferred_element_type=jnp.float32)
    h0 = jnp.maximum(acc + b_ref[0][None, :], 0.0)
    ns = lax.rsqrt(jnp.maximum(dout_ref[...], 1.0))
    h0_ref[...] = h0
    hs0_ref[...] = h0 * ns


def _tc_fc1(x, fc1_w, fc1_b, deg_out):
    return pl.pallas_call(
        _fc1_body,
        grid=(GRID,),
        in_specs=[
            pl.BlockSpec((BM, D), lambda i: (i, 0)),
            pl.BlockSpec((D, D), lambda i: (0, 0)),
            pl.BlockSpec((1, D), lambda i: (0, 0)),
            pl.BlockSpec((BM, 1), lambda i: (i, 0)),
        ],
        out_specs=[
            pl.BlockSpec((BM, D), lambda i: (i, 0)),
            pl.BlockSpec((BM, D), lambda i: (i, 0)),
        ],
        out_shape=[
            jax.ShapeDtypeStruct((N, D), jnp.float32),
            jax.ShapeDtypeStruct((N, D), jnp.float32),
        ],
    )(x, fc1_w, fc1_b, deg_out)


def _layer_body(beta, p0_ref, p1_ref, h0_ref, h_ref, din_ref, dout_ref, w_ref,
                hn_ref, hsn_ref):
    nd = lax.rsqrt(jnp.maximum(din_ref[...], 1.0))
    feat = (p0_ref[...] + p1_ref[...]) * nd
    feat = (1.0 - ALPHA_C) * feat + ALPHA_C * h0_ref[...]
    mm = jnp.dot(feat, w_ref[...], preferred_element_type=jnp.float32)
    rst = (1.0 - beta) * feat + beta * mm
    hn = jnp.maximum(rst + h_ref[...], 0.0)
    ns = lax.rsqrt(jnp.maximum(dout_ref[...], 1.0))
    hn_ref[...] = hn
    hsn_ref[...] = hn * ns


def _tc_layer(beta, p0, p1, h0, h, deg_in, deg_out, w):
    return pl.pallas_call(
        functools.partial(_layer_body, beta),
        grid=(GRID,),
        in_specs=[
            pl.BlockSpec((BM, D), lambda i: (i, 0)),
            pl.BlockSpec((BM, D), lambda i: (i, 0)),
            pl.BlockSpec((BM, D), lambda i: (i, 0)),
            pl.BlockSpec((BM, D), lambda i: (i, 0)),
            pl.BlockSpec((BM, 1), lambda i: (i, 0)),
            pl.BlockSpec((BM, 1), lambda i: (i, 0)),
            pl.BlockSpec((D, D), lambda i: (0, 0)),
        ],
        out_specs=[
            pl.BlockSpec((BM, D), lambda i: (i, 0)),
            pl.BlockSpec((BM, D), lambda i: (i, 0)),
        ],
        out_shape=[
            jax.ShapeDtypeStruct((N, D), jnp.float32),
            jax.ShapeDtypeStruct((N, D), jnp.float32),
        ],
    )(p0, p1, h0, h, deg_in, deg_out, w)


def _fc2_body(h_ref, w_ref, b_ref, o_ref):
    acc = jnp.dot(h_ref[...], w_ref[...], preferred_element_type=jnp.float32)
    o_ref[...] = acc + b_ref[0][None, :]


def _tc_fc2(h, fc2_w, fc2_b):
    return pl.pallas_call(
        _fc2_body,
        grid=(GRID,),
        in_specs=[
            pl.BlockSpec((BM, D), lambda i: (i, 0)),
            pl.BlockSpec((D, DC), lambda i: (0, 0)),
            pl.BlockSpec((1, DC), lambda i: (0, 0)),
        ],
        out_specs=pl.BlockSpec((BM, DC), lambda i: (i, 0)),
        out_shape=jax.ShapeDtypeStruct((N, DC), jnp.float32),
    )(h, fc2_w, fc2_b)


def _sc_copy(xin):
    nrows = xin.shape[0]
    rpw = nrows // NW

    @functools.partial(
        pl.kernel,
        out_type=jax.ShapeDtypeStruct(xin.shape, jnp.float32),
        mesh=_sc_mesh(),
        scratch_types=[
            pltpu.VMEM((rpw, 128), jnp.float32),
            pltpu.VMEM_SHARED((nrows, 128), jnp.float32),
        ],
    )
    def k(in_hbm, out_hbm, buf, sh):
        c = lax.axis_index("c")
        s = lax.axis_index("s")
        wid = c * NS + s

        pltpu.sync_copy(in_hbm.at[pl.ds(wid * rpw, rpw)], buf)
        pltpu.sync_copy(buf, sh.at[pl.ds(wid * rpw, rpw)])
        pltpu.sync_copy(sh.at[pl.ds(wid * rpw, rpw)],
                        out_hbm.at[pl.ds(wid * rpw, rpw)])

    return k(xin)


def kernel(x, edge_index, fc1_w, fc1_b, Ws, fc2_w, fc2_b):
    src = edge_index[0]
    dst = edge_index[1]

    src_p = jnp.concatenate([src, jnp.zeros((PAD,), jnp.int32)]).reshape(
        NW * CPT, CH)
    dst_p = jnp.concatenate([dst, jnp.full((PAD,), N, jnp.int32)]).reshape(
        NW * CPT, CH)
    ei_deg = jnp.concatenate(
        [edge_index, jnp.full((2, PAD), N, jnp.int32)], axis=1).reshape(
        2 * DEG_ROWS_TOTAL, CH)

    dj_out = jnp.bincount(src, length=N).astype(jnp.float32)
    dj_in = jnp.bincount(dst, length=N).astype(jnp.float32)
    dcat = jnp.broadcast_to(
        jnp.concatenate([dj_out, dj_in])[:, None], (2 * N, 16))
    dcat = jnp.concatenate(
        [dcat, jnp.zeros((480, 16), jnp.float32)]).reshape(2560, 128)
    degs = _sc_copy(dcat).reshape(20480, 16)
    deg_out = degs[:N, 0:1]
    deg_in = degs[N:2 * N, 0:1]

    h0, hs = _tc_fc1(x, fc1_w, fc1_b.reshape(1, D), deg_out)
    h = h0
    for l in range(NLAYERS):
        part = _sc_propagate(hs, src_p, dst_p)
        beta = math.log(LAMB_C / (l + 1) + 1.0)
        h, hs = _tc_layer(beta, part[:N], part[AGG_ROWS:AGG_ROWS + N], h0, h,
                          deg_in, deg_out, Ws[l])
    return _tc_fc2(h, fc2_w, fc2_b.reshape(1, DC))


# speedup vs baseline: 8.1734x; 3.2172x over previous
import functools
import math

import jax
import jax.numpy as jnp
from jax import lax
from jax.experimental import pallas as pl
from jax.experimental.pallas import tpu as pltpu
from jax.experimental.pallas import tpu_sc as plsc

N = 10000
E = 320000
D = 128
DC = 64
NLAYERS = 4
ALPHA_C = 0.5
LAMB_C = 1.0

NC = 2
NS = 16
NW = NC * NS
CH = 128
CPT = 80
WIN = 40
EP = NW * CPT * CH
PAD = EP - E
AGG_ROWS = 10112
ZPT = AGG_ROWS // NS

DEG_ROWS_TOTAL = EP // CH
DEG_RPT = DEG_ROWS_TOTAL // NS

BM = 1000
GRID = N // BM


def _sc_mesh():
    return plsc.VectorSubcoreMesh(core_axis_name="c", subcore_axis_name="s")


def _sc_degrees(ei_flat):
    @functools.partial(
        pl.kernel,
        out_type=jax.ShapeDtypeStruct((NC * AGG_ROWS, 16), jnp.float32),
        mesh=_sc_mesh(),
        scratch_types=[
            pltpu.VMEM((DEG_RPT, CH), jnp.int32),
            pltpu.VMEM((CH, 16), jnp.float32),
            pltpu.VMEM((CH, 16), jnp.float32),
            pltpu.VMEM_SHARED((AGG_ROWS, 16), jnp.float32),
        ],
    )
    def k(ei_hbm, out_hbm, idx_v, ones_v, zero_v, hist_sh):
        c = lax.axis_index("c")
        s = lax.axis_index("s")

        @pl.loop(0, CH)
        def _(i):
            ones_v[i, pl.ds(0, 16)] = jnp.ones((16,), jnp.float32)
            zero_v[i, pl.ds(0, 16)] = jnp.zeros((16,), jnp.float32)

        zbase = s * ZPT
        for kk in range(ZPT // CH):
            pltpu.sync_copy(zero_v, hist_sh.at[pl.ds(zbase + kk * CH, CH)])
        rem = ZPT % CH
        pltpu.sync_copy(zero_v.at[pl.ds(0, rem)],
                        hist_sh.at[pl.ds(zbase + (ZPT // CH) * CH, rem)])

        pltpu.sync_copy(
            ei_hbm.at[pl.ds(c * DEG_ROWS_TOTAL + s * DEG_RPT, DEG_RPT)], idx_v)
        plsc.subcore_barrier()

        @pl.loop(0, DEG_RPT)
        def _(j):
            pltpu.sync_copy(ones_v, hist_sh.at[idx_v.at[j]], add=True)

        plsc.subcore_barrier()
        pltpu.sync_copy(hist_sh.at[pl.ds(zbase, ZPT)],
                        out_hbm.at[pl.ds(c * AGG_ROWS + zbase, ZPT)])

    return k(ei_flat)


def _sc_propagate(hs, src2d, dst2d):
    @functools.partial(
        pl.kernel,
        out_type=jax.ShapeDtypeStruct((NC * AGG_ROWS, D), jnp.float32),
        mesh=_sc_mesh(),
        scratch_types=[
            pltpu.VMEM((WIN, CH), jnp.int32),
            pltpu.VMEM((WIN, CH), jnp.int32),
            pltpu.VMEM((CH, D), jnp.float32),
            pltpu.VMEM((CH, D), jnp.float32),
            pltpu.VMEM_SHARED((AGG_ROWS, D), jnp.float32),
            pltpu.SemaphoreType.DMA,
            pltpu.SemaphoreType.DMA,
        ],
    )
    def k(hs_hbm, src_hbm, dst_hbm, out_hbm, src_v, dst_v, bufa, bufb, agg_sh,
          sema, semb):
        c = lax.axis_index("c")
        s = lax.axis_index("s")
        wid = c * NS + s

        @pl.loop(0, CH)
        def _(i):
            for kk in range(D // 16):
                bufa[i, pl.ds(kk * 16, 16)] = jnp.zeros((16,), jnp.float32)

        zbase = s * ZPT
        for kk in range(ZPT // CH):
            pltpu.sync_copy(bufa, agg_sh.at[pl.ds(zbase + kk * CH, CH)])
        rem = ZPT % CH
        pltpu.sync_copy(bufa.at[pl.ds(0, rem)],
                        agg_sh.at[pl.ds(zbase + (ZPT // CH) * CH, rem)])

        row0 = wid * CPT
        pltpu.sync_copy(src_hbm.at[pl.ds(row0, WIN)], src_v)
        pltpu.sync_copy(dst_hbm.at[pl.ds(row0, WIN)], dst_v)
        plsc.subcore_barrier()

        def wrow(k):
            return k - jnp.where(k >= WIN, WIN, 0)

        def start_g(k, buf, sem):
            pltpu.async_copy(hs_hbm.at[src_v.at[wrow(k)]], buf, sem)

        def wait_g(buf, sem):
            pltpu.make_async_copy(hs_hbm.at[src_v.at[0]], buf, sem).wait()

        def scat(k, buf):
            pltpu.sync_copy(buf, agg_sh.at[dst_v.at[wrow(k)]], add=True)

        start_g(0, bufa, sema)

        @pl.loop(0, CPT - 2, step=2)
        def _(j):
            wait_g(bufa, sema)
            start_g(j + 1, bufb, semb)
            scat(j, bufa)
            wait_g(bufb, semb)

            @pl.when(j == WIN - 2)
            def _():
                pltpu.sync_copy(src_hbm.at[pl.ds(row0 + WIN, WIN)], src_v)

            start_g(j + 2, bufa, sema)
            scat(j + 1, bufb)

            @pl.when(j == WIN - 2)
            def _():
                pltpu.sync_copy(dst_hbm.at[pl.ds(row0 + WIN, WIN)], dst_v)

        wait_g(bufa, sema)
        start_g(CPT - 1, bufb, semb)
        scat(CPT - 2, bufa)
        wait_g(bufb, semb)
        scat(CPT - 1, bufb)

        plsc.subcore_barrier()
        pltpu.sync_copy(agg_sh.at[pl.ds(zbase, ZPT)],
                        out_hbm.at[pl.ds(c * AGG_ROWS + zbase, ZPT)])

    return k(hs, src2d, dst2d)


def _fc1_body(x_ref, w_ref, b_ref, dout_ref, h0_ref, hs0_ref):
    acc = jnp.dot(x_ref[...], w_ref[...], preferred_element_type=jnp.float32)
    h0 = jnp.maximum(acc + b_ref[0][None, :], 0.0)
    ns = lax.rsqrt(jnp.maximum(dout_ref[...], 1.0))
    h0_ref[...] = h0
    hs0_ref[...] = h0 * ns


def _tc_fc1(x, fc1_w, fc1_b, deg_out):
    return pl.pallas_call(
        _fc1_body,
        grid=(GRID,),
        in_specs=[
            pl.BlockSpec((BM, D), lambda i: (i, 0)),
            pl.BlockSpec((D, D), lambda i: (0, 0)),
            pl.BlockSpec((1, D), lambda i: (0, 0)),
            pl.BlockSpec((BM, 1), lambda i: (i, 0)),
        ],
        out_specs=[
            pl.BlockSpec((BM, D), lambda i: (i, 0)),
            pl.BlockSpec((BM, D), lambda i: (i, 0)),
        ],
        out_shape=[
            jax.ShapeDtypeStruct((N, D), jnp.float32),
            jax.ShapeDtypeStruct((N, D), jnp.float32),
        ],
    )(x, fc1_w, fc1_b, deg_out)


def _layer_body(beta, p0_ref, p1_ref, h0_ref, h_ref, din_ref, dout_ref, w_ref,
                hn_ref, hsn_ref):
    nd = lax.rsqrt(jnp.maximum(din_ref[...], 1.0))
    feat = (p0_ref[...] + p1_ref[...]) * nd
    feat = (1.0 - ALPHA_C) * feat + ALPHA_C * h0_ref[...]
    mm = jnp.dot(feat, w_ref[...], preferred_element_type=jnp.float32)
    rst = (1.0 - beta) * feat + beta * mm
    hn = jnp.maximum(rst + h_ref[...], 0.0)
    ns = lax.rsqrt(jnp.maximum(dout_ref[...], 1.0))
    hn_ref[...] = hn
    hsn_ref[...] = hn * ns


def _tc_layer(beta, p0, p1, h0, h, deg_in, deg_out, w):
    return pl.pallas_call(
        functools.partial(_layer_body, beta),
        grid=(GRID,),
        in_specs=[
            pl.BlockSpec((BM, D), lambda i: (i, 0)),
            pl.BlockSpec((BM, D), lambda i: (i, 0)),
            pl.BlockSpec((BM, D), lambda i: (i, 0)),
            pl.BlockSpec((BM, D), lambda i: (i, 0)),
            pl.BlockSpec((BM, 1), lambda i: (i, 0)),
            pl.BlockSpec((BM, 1), lambda i: (i, 0)),
            pl.BlockSpec((D, D), lambda i: (0, 0)),
        ],
        out_specs=[
            pl.BlockSpec((BM, D), lambda i: (i, 0)),
            pl.BlockSpec((BM, D), lambda i: (i, 0)),
        ],
        out_shape=[
            jax.ShapeDtypeStruct((N, D), jnp.float32),
            jax.ShapeDtypeStruct((N, D), jnp.float32),
        ],
    )(p0, p1, h0, h, deg_in, deg_out, w)


def _fc2_body(h_ref, w_ref, b_ref, o_ref):
    acc = jnp.dot(h_ref[...], w_ref[...], preferred_element_type=jnp.float32)
    o_ref[...] = acc + b_ref[0][None, :]


def _tc_fc2(h, fc2_w, fc2_b):
    return pl.pallas_call(
        _fc2_body,
        grid=(GRID,),
        in_specs=[
            pl.BlockSpec((BM, D), lambda i: (i, 0)),
            pl.BlockSpec((D, DC), lambda i: (0, 0)),
            pl.BlockSpec((1, DC), lambda i: (0, 0)),
        ],
        out_specs=pl.BlockSpec((BM, DC), lambda i: (i, 0)),
        out_shape=jax.ShapeDtypeStruct((N, DC), jnp.float32),
    )(h, fc2_w, fc2_b)


def _sc_copy(xin):
    nrows = xin.shape[0]
    rpw = nrows // NW

    @functools.partial(
        pl.kernel,
        out_type=jax.ShapeDtypeStruct(xin.shape, jnp.float32),
        mesh=_sc_mesh(),
        scratch_types=[
            pltpu.VMEM((rpw, 128), jnp.float32),
            pltpu.VMEM_SHARED((nrows, 128), jnp.float32),
        ],
    )
    def k(in_hbm, out_hbm, buf, sh):
        c = lax.axis_index("c")
        s = lax.axis_index("s")
        wid = c * NS + s

        pltpu.sync_copy(in_hbm.at[pl.ds(wid * rpw, rpw)], buf)
        pltpu.sync_copy(buf, sh.at[pl.ds(wid * rpw, rpw)])
        pltpu.sync_copy(sh.at[pl.ds(wid * rpw, rpw)],
                        out_hbm.at[pl.ds(wid * rpw, rpw)])

    return k(xin)


def kernel(x, edge_index, fc1_w, fc1_b, Ws, fc2_w, fc2_b):
    src = edge_index[0]
    dst = edge_index[1]

    pad_src = jnp.arange(PAD, dtype=jnp.int32) % N
    pad_dst = N + jnp.arange(PAD, dtype=jnp.int32) % (AGG_ROWS - N)
    src_p = jnp.concatenate([src, pad_src]).reshape(NW * CPT, CH)
    dst_p = jnp.concatenate([dst, pad_dst]).reshape(NW * CPT, CH)
    ei_deg = jnp.concatenate(
        [edge_index, jnp.full((2, PAD), N, jnp.int32)], axis=1).reshape(
        2 * DEG_ROWS_TOTAL, CH)

    dj_out = jnp.bincount(src, length=N).astype(jnp.float32)
    dj_in = jnp.bincount(dst, length=N).astype(jnp.float32)
    dcat = jnp.broadcast_to(
        jnp.concatenate([dj_out, dj_in])[:, None], (2 * N, 16))
    dcat = jnp.concatenate(
        [dcat, jnp.zeros((480, 16), jnp.float32)]).reshape(2560, 128)
    degs = _sc_copy(dcat).reshape(20480, 16)
    deg_out = degs[:N, 0:1]
    deg_in = degs[N:2 * N, 0:1]

    h0, hs = _tc_fc1(x, fc1_w, fc1_b.reshape(1, D), deg_out)
    h = h0
    for l in range(NLAYERS):
        part = _sc_propagate(hs, src_p, dst_p)
        beta = math.log(LAMB_C / (l + 1) + 1.0)
        h, hs = _tc_layer(beta, part[:N], part[AGG_ROWS:AGG_ROWS + N], h0, h,
                          deg_in, deg_out, Ws[l])
    return _tc_fc2(h, fc2_w, fc2_b.reshape(1, DC))


# speedup vs baseline: 8.6533x; 1.0587x over previous
import dataclasses
import functools
import math

import jax
import jax.numpy as jnp
from jax import lax
from jax.experimental import pallas as pl
from jax.experimental.pallas import tpu as pltpu
from jax.experimental.pallas import tpu_sc as plsc

N = 10000
E = 320000
D = 128
DC = 64
NLAYERS = 4
ALPHA_C = 0.5
LAMB_C = 1.0

NC = 2
NS = 16
NW = NC * NS
CH = 128
CPT = 80
WIN = 40
EP = NW * CPT * CH
PAD = EP - E
AGG_ROWS = 10112
ZPT = AGG_ROWS // NS

DEG_ROWS_TOTAL = EP // CH
DEG_RPT = DEG_ROWS_TOTAL // NS
DEGN = 10240
DEG_SEG = DEGN // NS

BM = 1000
GRID = N // BM


def _sc_mesh():
    return plsc.VectorSubcoreMesh(core_axis_name="c", subcore_axis_name="s")


def _sc_params():
    cp = pltpu.CompilerParams()
    if "needs_layout_passes" in pltpu.CompilerParams.__dataclass_fields__:
        cp = dataclasses.replace(cp, needs_layout_passes=False)
    return cp


def _sc_degrees(ei_flat):
    @functools.partial(
        pl.kernel,
        out_type=jax.ShapeDtypeStruct((NC * DEGN,), jnp.float32),
        mesh=_sc_mesh(),
        compiler_params=_sc_params(),
        scratch_types=[
            pltpu.VMEM((DEG_RPT, CH), jnp.int32),
            pltpu.VMEM((DEGN,), jnp.float32),
            pltpu.VMEM((DEG_SEG,), jnp.float32),
            pltpu.VMEM((DEG_SEG,), jnp.float32),
            pltpu.VMEM_SHARED((NS, DEGN), jnp.float32),
        ],
    )
    def k(ei_hbm, out_hbm, idx_v, hist, acc, stg, sh):
        c = lax.axis_index("c")
        s = lax.axis_index("s")

        @pl.loop(0, DEGN, step=16)
        def _(i):
            hist[pl.ds(i, 16)] = jnp.zeros((16,), jnp.float32)

        pltpu.sync_copy(
            ei_hbm.at[pl.ds(c * DEG_ROWS_TOTAL + s * DEG_RPT, DEG_RPT)], idx_v)
        ones = jnp.ones((16,), jnp.float32)

        @pl.loop(0, DEG_RPT)
        def _(r):
            for kk in range(CH // 16):
                iv = idx_v[r, pl.ds(kk * 16, 16)]
                plsc.addupdate_scatter(hist, [iv], ones)

        pltpu.sync_copy(hist, sh.at[s])
        plsc.subcore_barrier()

        @pl.loop(0, DEG_SEG, step=16)
        def _(i):
            acc[pl.ds(i, 16)] = jnp.zeros((16,), jnp.float32)

        for r in range(NS):
            pltpu.sync_copy(sh.at[r, pl.ds(s * DEG_SEG, DEG_SEG)], stg)

            @pl.loop(0, DEG_SEG, step=16)
            def _(i):
                acc[pl.ds(i, 16)] = acc[pl.ds(i, 16)] + stg[pl.ds(i, 16)]

        pltpu.sync_copy(acc, out_hbm.at[pl.ds(c * DEGN + s * DEG_SEG, DEG_SEG)])

    return k(ei_flat)


def _sc_propagate(hs, src2d, dst2d):
    @functools.partial(
        pl.kernel,
        out_type=jax.ShapeDtypeStruct((NC * AGG_ROWS, D), jnp.float32),
        mesh=_sc_mesh(),
        scratch_types=[
            pltpu.VMEM((WIN, CH), jnp.int32),
            pltpu.VMEM((WIN, CH), jnp.int32),
            pltpu.VMEM((CH, D), jnp.float32),
            pltpu.VMEM((CH, D), jnp.float32),
            pltpu.VMEM_SHARED((AGG_ROWS, D), jnp.float32),
            pltpu.SemaphoreType.DMA,
            pltpu.SemaphoreType.DMA,
        ],
    )
    def k(hs_hbm, src_hbm, dst_hbm, out_hbm, src_v, dst_v, bufa, bufb, agg_sh,
          sema, semb):
        c = lax.axis_index("c")
        s = lax.axis_index("s")
        wid = c * NS + s

        @pl.loop(0, CH)
        def _(i):
            for kk in range(D // 16):
                bufa[i, pl.ds(kk * 16, 16)] = jnp.zeros((16,), jnp.float32)

        zbase = s * ZPT
        for kk in range(ZPT // CH):
            pltpu.sync_copy(bufa, agg_sh.at[pl.ds(zbase + kk * CH, CH)])
        rem = ZPT % CH
        pltpu.sync_copy(bufa.at[pl.ds(0, rem)],
                        agg_sh.at[pl.ds(zbase + (ZPT // CH) * CH, rem)])

        row0 = wid * CPT
        pltpu.sync_copy(src_hbm.at[pl.ds(row0, WIN)], src_v)
        pltpu.sync_copy(dst_hbm.at[pl.ds(row0, WIN)], dst_v)
        plsc.subcore_barrier()

        def wrow(k):
            return k - jnp.where(k >= WIN, WIN, 0)

        def start_g(k, buf, sem):
            pltpu.async_copy(hs_hbm.at[src_v.at[wrow(k)]], buf, sem)

        def wait_g(buf, sem):
            pltpu.make_async_copy(hs_hbm.at[src_v.at[0]], buf, sem).wait()

        def scat(k, buf):
            pltpu.sync_copy(buf, agg_sh.at[dst_v.at[wrow(k)]], add=True)

        start_g(0, bufa, sema)

        @pl.loop(0, CPT - 2, step=2)
        def _(j):
            wait_g(bufa, sema)
            start_g(j + 1, bufb, semb)
            scat(j, bufa)
            wait_g(bufb, semb)

            @pl.when(j == WIN - 2)
            def _():
                pltpu.sync_copy(src_hbm.at[pl.ds(row0 + WIN, WIN)], src_v)

            start_g(j + 2, bufa, sema)
            scat(j + 1, bufb)

            @pl.when(j == WIN - 2)
            def _():
                pltpu.sync_copy(dst_hbm.at[pl.ds(row0 + WIN, WIN)], dst_v)

        wait_g(bufa, sema)
        start_g(CPT - 1, bufb, semb)
        scat(CPT - 2, bufa)
        wait_g(bufb, semb)
        scat(CPT - 1, bufb)

        plsc.subcore_barrier()
        pltpu.sync_copy(agg_sh.at[pl.ds(zbase, ZPT)],
                        out_hbm.at[pl.ds(c * AGG_ROWS + zbase, ZPT)])

    return k(hs, src2d, dst2d)


def _fc1_body(x_ref, w_ref, b_ref, dout_ref, h0_ref, hs0_ref):
    acc = jnp.dot(x_ref[...], w_ref[...], preferred_element_type=jnp.float32)
    h0 = jnp.maximum(acc + b_ref[0][None, :], 0.0)
    ns = lax.rsqrt(jnp.maximum(dout_ref[...], 1.0))
    h0_ref[...] = h0
    hs0_ref[...] = h0 * ns


def _tc_fc1(x, fc1_w, fc1_b, deg_out):
    return pl.pallas_call(
        _fc1_body,
        grid=(GRID,),
        in_specs=[
            pl.BlockSpec((BM, D), lambda i: (i, 0)),
            pl.BlockSpec((D, D), lambda i: (0, 0)),
            pl.BlockSpec((1, D), lambda i: (0, 0)),
            pl.BlockSpec((BM, 1), lambda i: (i, 0)),
        ],
        out_specs=[
            pl.BlockSpec((BM, D), lambda i: (i, 0)),
            pl.BlockSpec((BM, D), lambda i: (i, 0)),
        ],
        out_shape=[
            jax.ShapeDtypeStruct((N, D), jnp.float32),
            jax.ShapeDtypeStruct((N, D), jnp.float32),
        ],
    )(x, fc1_w, fc1_b, deg_out)


def _layer_body(beta, p0_ref, p1_ref, h0_ref, h_ref, din_ref, dout_ref, w_ref,
                hn_ref, hsn_ref):
    nd = lax.rsqrt(jnp.maximum(din_ref[...], 1.0))
    feat = (p0_ref[...] + p1_ref[...]) * nd
    feat = (1.0 - ALPHA_C) * feat + ALPHA_C * h0_ref[...]
    mm = jnp.dot(feat, w_ref[...], preferred_element_type=jnp.float32)
    rst = (1.0 - beta) * feat + beta * mm
    hn = jnp.maximum(rst + h_ref[...], 0.0)
    ns = lax.rsqrt(jnp.maximum(dout_ref[...], 1.0))
    hn_ref[...] = hn
    hsn_ref[...] = hn * ns


def _tc_layer(beta, p0, p1, h0, h, deg_in, deg_out, w):
    return pl.pallas_call(
        functools.partial(_layer_body, beta),
        grid=(GRID,),
        in_specs=[
            pl.BlockSpec((BM, D), lambda i: (i, 0)),
            pl.BlockSpec((BM, D), lambda i: (i, 0)),
            pl.BlockSpec((BM, D), lambda i: (i, 0)),
            pl.BlockSpec((BM, D), lambda i: (i, 0)),
            pl.BlockSpec((BM, 1), lambda i: (i, 0)),
            pl.BlockSpec((BM, 1), lambda i: (i, 0)),
            pl.BlockSpec((D, D), lambda i: (0, 0)),
        ],
        out_specs=[
            pl.BlockSpec((BM, D), lambda i: (i, 0)),
            pl.BlockSpec((BM, D), lambda i: (i, 0)),
        ],
        out_shape=[
            jax.ShapeDtypeStruct((N, D), jnp.float32),
            jax.ShapeDtypeStruct((N, D), jnp.float32),
        ],
    )(p0, p1, h0, h, deg_in, deg_out, w)


def _fc2_body(h_ref, w_ref, b_ref, o_ref):
    acc = jnp.dot(h_ref[...], w_ref[...], preferred_element_type=jnp.float32)
    o_ref[...] = acc + b_ref[0][None, :]


def _tc_fc2(h, fc2_w, fc2_b):
    return pl.pallas_call(
        _fc2_body,
        grid=(GRID,),
        in_specs=[
            pl.BlockSpec((BM, D), lambda i: (i, 0)),
            pl.BlockSpec((D, DC), lambda i: (0, 0)),
            pl.BlockSpec((1, DC), lambda i: (0, 0)),
        ],
        out_specs=pl.BlockSpec((BM, DC), lambda i: (i, 0)),
        out_shape=jax.ShapeDtypeStruct((N, DC), jnp.float32),
    )(h, fc2_w, fc2_b)


def kernel(x, edge_index, fc1_w, fc1_b, Ws, fc2_w, fc2_b):
    src = edge_index[0]
    dst = edge_index[1]

    pad_src = jnp.arange(PAD, dtype=jnp.int32) % N
    pad_dst = N + jnp.arange(PAD, dtype=jnp.int32) % (AGG_ROWS - N)
    src_p = jnp.concatenate([src, pad_src]).reshape(NW * CPT, CH)
    dst_p = jnp.concatenate([dst, pad_dst]).reshape(NW * CPT, CH)
    pad_deg = N + jnp.arange(PAD, dtype=jnp.int32) % (DEGN - N)
    ei_deg = jnp.concatenate(
        [edge_index, jnp.stack([pad_deg, pad_deg])], axis=1).reshape(
        2 * DEG_ROWS_TOTAL, CH)

    degs = _sc_degrees(ei_deg)
    deg_out = degs[:N, None]
    deg_in = degs[DEGN:DEGN + N, None]

    h0, hs = _tc_fc1(x, fc1_w, fc1_b.reshape(1, D), deg_out)
    h = h0
    for l in range(NLAYERS):
        part = _sc_propagate(hs, src_p, dst_p)
        beta = math.log(LAMB_C / (l + 1) + 1.0)
        h, hs = _tc_layer(beta, part[:N], part[AGG_ROWS:AGG_ROWS + N], h0, h,
                          deg_in, deg_out, Ws[l])
    return _tc_fc2(h, fc2_w, fc2_b.reshape(1, DC))
